# baseline (speedup 1.0000x reference)
import functools

import jax
import jax.numpy as jnp
from jax import lax
from jax.experimental import pallas as pl
from jax.experimental.pallas import tpu as pltpu
from jax.experimental.pallas import tpu_sc as plsc

_NC = 2
_NS = 16
_NW = _NC * _NS
_LANE = 16

_D1 = 18
_D2 = 15
_GW = 128
_HALF = 256
_CHUNK = 128


def _sc_gather(idxs, t1, t2, B):
    b_per_w = B // _NW
    mesh = plsc.VectorSubcoreMesh(core_axis_name="c", subcore_axis_name="s")

    @functools.partial(
        pl.kernel,
        mesh=mesh,
        out_type=(
            jax.ShapeDtypeStruct((B, _GW), jnp.float32),
            jax.ShapeDtypeStruct((B, _GW), jnp.float32),
        ),
        scratch_types=[
            pltpu.VMEM((6, b_per_w), jnp.int32),
            pltpu.VMEM((b_per_w,), jnp.int32),
            pltpu.VMEM((b_per_w,), jnp.int32),
            pltpu.VMEM((_HALF, _GW), jnp.float32),
            pltpu.SemaphoreType.DMA,
        ],
    )
    def gather_k(i0, i1, i2, i3, i4, i5, t1_hbm, t2_hbm, out1, out2,
                 idx_raw, idx1, idx2, rows, sem):
        wid = lax.axis_index("s") * _NC + lax.axis_index("c")
        base = wid * b_per_w
        for f, i_hbm in enumerate((i0, i1, i2, i3, i4, i5)):
            pltpu.sync_copy(i_hbm.at[wid], idx_raw.at[f])

        @pl.loop(0, b_per_w // _LANE)
        def combine(g):
            sl = pl.ds(g * _LANE, _LANE)
            st = idx_raw[0, sl]
            yr = idx_raw[1, sl]
            mo = idx_raw[2, sl]
            dw = idx_raw[3, sl]
            hr = idx_raw[4, sl]
            se = idx_raw[5, sl]
            idx1[sl] = (st * 48 + yr * 12) + mo
            idx2[sl] = (dw * 96 + hr * 4) + se

        for h in range(b_per_w // _HALF):
            off = h * _HALF
            for tab, idxv, out in ((t1_hbm, idx1, out1), (t2_hbm, idx2, out2)):
                cps = [
                    pltpu.async_copy(
                        tab.at[idxv.at[pl.ds(off + c * _CHUNK, _CHUNK)]],
                        rows.at[pl.ds(c * _CHUNK, _CHUNK)], sem)
                    for c in range(_HALF // _CHUNK)
                ]
                for cp in cps:
                    cp.wait()
                pltpu.sync_copy(rows, out.at[pl.ds(base + off, _HALF)])

    return gather_k(*idxs, t1, t2)


def _mlp_body(e1, e2, w1, b1, w2, b2, w3, b3, w4, b4, w5, b5, out_ref):
    x = jnp.concatenate([e1[...], e2[...]], axis=1).astype(jnp.bfloat16)
    h = jnp.dot(x, w1[...], preferred_element_type=jnp.float32) + b1[...]
    h = jnp.maximum(h, 0.0).astype(jnp.bfloat16)
    h = jnp.dot(h, w2[...], preferred_element_type=jnp.float32) + b2[...]
    h = jnp.maximum(h, 0.0).astype(jnp.bfloat16)
    h = jnp.dot(h, w3[...], preferred_element_type=jnp.float32) + b3[...]
    h = jnp.maximum(h, 0.0).astype(jnp.bfloat16)
    h = jnp.dot(h, w4[...], preferred_element_type=jnp.float32) + b4[...]
    h = jnp.maximum(h, 0.0).astype(jnp.bfloat16)
    z = jnp.dot(h, w5[...], preferred_element_type=jnp.float32) + b5[...]
    out_ref[...] = 1.0 / (1.0 + jnp.exp(-z))


def _mlp(e1, e2, w1p, b1, w2, b2, w3, b3, w4, b4, w5, b5, tile=2048):
    B = e1.shape[0]
    full = lambda arr: pl.BlockSpec(arr.shape, lambda i: (0,) * arr.ndim)
    return pl.pallas_call(
        _mlp_body,
        grid=(B // tile,),
        in_specs=[
            pl.BlockSpec((tile, _GW), lambda i: (i, 0)),
            pl.BlockSpec((tile, _GW), lambda i: (i, 0)),
            full(w1p), full(b1), full(w2), full(b2),
            full(w3), full(b3), full(w4), full(b4),
            full(w5), full(b5),
        ],
        out_specs=pl.BlockSpec((tile, 1), lambda i: (i, 0)),
        out_shape=jax.ShapeDtypeStruct((B, 1), jnp.float32),
    )(e1, e2, w1p, b1, w2, b2, w3, b3, w4, b4, w5, b5)


def kernel(stations, year, month, day_of_week, hour, season,
           E_st, E_yr, E_mo, E_dw, E_hr, E_se,
           W1, b1, W2, b2, W3, b3, W4, b4, W5, b5):
    B = stations.shape[0]

    t1 = jnp.concatenate([
        jnp.broadcast_to(E_st[:, None, None, :], (33, 4, 12, 10)),
        jnp.broadcast_to(E_yr[None, :, None, :], (33, 4, 12, 2)),
        jnp.broadcast_to(E_mo[None, None, :, :], (33, 4, 12, 6)),
    ], axis=-1).reshape(1584, _D1)
    t1 = jnp.pad(t1, ((0, 0), (0, _GW - _D1)))
    t2 = jnp.concatenate([
        jnp.broadcast_to(E_dw[:, None, None, :], (7, 24, 4, 3)),
        jnp.broadcast_to(E_hr[None, :, None, :], (7, 24, 4, 10)),
        jnp.broadcast_to(E_se[None, None, :, :], (7, 24, 4, 2)),
    ], axis=-1).reshape(672, _D2)
    t2 = jnp.pad(t2, ((0, 0), (0, _GW - _D2)))

    bf = jnp.bfloat16
    w1p = jnp.zeros((2 * _GW, W1.shape[1]), bf)
    w1p = lax.dynamic_update_slice(w1p, W1[:_D1].astype(bf), (0, 0))
    w1p = lax.dynamic_update_slice(w1p, W1[_D1:].astype(bf), (_GW, 0))

    b_per_w = B // _NW
    idxs = [a.astype(jnp.int32).reshape(_NW, b_per_w) for a in
            (stations, year, month, day_of_week, hour, season)]
    e1, e2 = _sc_gather(idxs, t1, t2, B)

    return _mlp(e1, e2,
                w1p, b1.reshape(1, -1),
                W2.astype(bf), b2.reshape(1, -1),
                W3.astype(bf), b3.reshape(1, -1),
                W4.astype(bf), b4.reshape(1, -1),
                W5.astype(bf), b5.reshape(1, -1))


# speedup vs baseline: 1.0217x; 1.0217x over previous
import functools

import jax
import jax.numpy as jnp
from jax import lax
from jax.experimental import pallas as pl
from jax.experimental.pallas import tpu as pltpu
from jax.experimental.pallas import tpu_sc as plsc

_NC = 2
_NS = 16
_NW = _NC * _NS
_LANE = 16

_D1 = 18
_D2 = 15
_GW = 128
_HALF = 512
_CHUNK = 128


def _sc_gather(idxs, t1, t2, B):
    b_per_w = B // _NW
    mesh = plsc.VectorSubcoreMesh(core_axis_name="c", subcore_axis_name="s")

    @functools.partial(
        pl.kernel,
        mesh=mesh,
        out_type=(
            jax.ShapeDtypeStruct((B, _GW), jnp.float32),
            jax.ShapeDtypeStruct((B, _GW), jnp.float32),
        ),
        scratch_types=[
            pltpu.VMEM((6, b_per_w), jnp.int32),
            pltpu.VMEM((b_per_w,), jnp.int32),
            pltpu.VMEM((b_per_w,), jnp.int32),
            pltpu.VMEM((_HALF, _GW), jnp.float32),
            pltpu.SemaphoreType.DMA,
        ],
    )
    def gather_k(i0, i1, i2, i3, i4, i5, t1_hbm, t2_hbm, out1, out2,
                 idx_raw, idx1, idx2, rows, sem):
        wid = lax.axis_index("s") * _NC + lax.axis_index("c")
        base = wid * b_per_w
        for f, i_hbm in enumerate((i0, i1, i2, i3, i4, i5)):
            pltpu.sync_copy(i_hbm.at[wid], idx_raw.at[f])

        @pl.loop(0, b_per_w // _LANE)
        def combine(g):
            sl = pl.ds(g * _LANE, _LANE)
            st = idx_raw[0, sl]
            yr = idx_raw[1, sl]
            mo = idx_raw[2, sl]
            dw = idx_raw[3, sl]
            hr = idx_raw[4, sl]
            se = idx_raw[5, sl]
            idx1[sl] = (st * 48 + yr * 12) + mo
            idx2[sl] = (dw * 96 + hr * 4) + se

        for h in range(b_per_w // _HALF):
            off = h * _HALF
            for tab, idxv, out in ((t1_hbm, idx1, out1), (t2_hbm, idx2, out2)):
                cps = [
                    pltpu.async_copy(
                        tab.at[idxv.at[pl.ds(off + c * _CHUNK, _CHUNK)]],
                        rows.at[pl.ds(c * _CHUNK, _CHUNK)], sem)
                    for c in range(_HALF // _CHUNK)
                ]
                for cp in cps:
                    cp.wait()
                pltpu.sync_copy(rows, out.at[pl.ds(base + off, _HALF)])

    return gather_k(*idxs, t1, t2)


def _mlp_body(e1, e2, w1, b1, w2, b2, w3, b3, w4, b4, w5, b5, out_ref):
    x = jnp.concatenate([e1[...], e2[...]], axis=1).astype(jnp.bfloat16)
    h = jnp.dot(x, w1[...], preferred_element_type=jnp.float32) + b1[...]
    h = jnp.maximum(h, 0.0).astype(jnp.bfloat16)
    h = jnp.dot(h, w2[...], preferred_element_type=jnp.float32) + b2[...]
    h = jnp.maximum(h, 0.0).astype(jnp.bfloat16)
    h = jnp.dot(h, w3[...], preferred_element_type=jnp.float32) + b3[...]
    h = jnp.maximum(h, 0.0).astype(jnp.bfloat16)
    h = jnp.dot(h, w4[...], preferred_element_type=jnp.float32) + b4[...]
    h = jnp.maximum(h, 0.0).astype(jnp.bfloat16)
    z = jnp.dot(h, w5[...], preferred_element_type=jnp.float32) + b5[...]
    out_ref[...] = 1.0 / (1.0 + jnp.exp(-z))


def _mlp(e1, e2, w1p, b1, w2, b2, w3, b3, w4, b4, w5, b5, tile=2048):
    B = e1.shape[0]
    full = lambda arr: pl.BlockSpec(arr.shape, lambda i: (0,) * arr.ndim)
    return pl.pallas_call(
        _mlp_body,
        grid=(B // tile,),
        in_specs=[
            pl.BlockSpec((tile, _GW), lambda i: (i, 0)),
            pl.BlockSpec((tile, _GW), lambda i: (i, 0)),
            full(w1p), full(b1), full(w2), full(b2),
            full(w3), full(b3), full(w4), full(b4),
            full(w5), full(b5),
        ],
        out_specs=pl.BlockSpec((tile, 1), lambda i: (i, 0)),
        out_shape=jax.ShapeDtypeStruct((B, 1), jnp.float32),
    )(e1, e2, w1p, b1, w2, b2, w3, b3, w4, b4, w5, b5)


def kernel(stations, year, month, day_of_week, hour, season,
           E_st, E_yr, E_mo, E_dw, E_hr, E_se,
           W1, b1, W2, b2, W3, b3, W4, b4, W5, b5):
    B = stations.shape[0]

    t1 = jnp.concatenate([
        jnp.broadcast_to(E_st[:, None, None, :], (33, 4, 12, 10)),
        jnp.broadcast_to(E_yr[None, :, None, :], (33, 4, 12, 2)),
        jnp.broadcast_to(E_mo[None, None, :, :], (33, 4, 12, 6)),
    ], axis=-1).reshape(1584, _D1)
    t1 = jnp.pad(t1, ((0, 0), (0, _GW - _D1)))
    t2 = jnp.concatenate([
        jnp.broadcast_to(E_dw[:, None, None, :], (7, 24, 4, 3)),
        jnp.broadcast_to(E_hr[None, :, None, :], (7, 24, 4, 10)),
        jnp.broadcast_to(E_se[None, None, :, :], (7, 24, 4, 2)),
    ], axis=-1).reshape(672, _D2)
    t2 = jnp.pad(t2, ((0, 0), (0, _GW - _D2)))

    bf = jnp.bfloat16
    w1p = jnp.zeros((2 * _GW, W1.shape[1]), bf)
    w1p = lax.dynamic_update_slice(w1p, W1[:_D1].astype(bf), (0, 0))
    w1p = lax.dynamic_update_slice(w1p, W1[_D1:].astype(bf), (_GW, 0))

    b_per_w = B // _NW
    idxs = [a.astype(jnp.int32).reshape(_NW, b_per_w) for a in
            (stations, year, month, day_of_week, hour, season)]
    e1, e2 = _sc_gather(idxs, t1, t2, B)

    return _mlp(e1, e2,
                w1p, b1.reshape(1, -1),
                W2.astype(bf), b2.reshape(1, -1),
                W3.astype(bf), b3.reshape(1, -1),
                W4.astype(bf), b4.reshape(1, -1),
                W5.astype(bf), b5.reshape(1, -1))


# speedup vs baseline: 1.0468x; 1.0246x over previous
import functools

import jax
import jax.numpy as jnp
from jax import lax
from jax.experimental import pallas as pl
from jax.experimental.pallas import tpu as pltpu
from jax.experimental.pallas import tpu_sc as plsc

_NC = 2
_NS = 16
_NW = _NC * _NS
_LANE = 16

_D1 = 18
_D2 = 15
_GW = 128
_HALF = 512
_CHUNK = 128


def _sc_gather(idxs, t1, t2, B):
    b_per_w = B // _NW
    mesh = plsc.VectorSubcoreMesh(core_axis_name="c", subcore_axis_name="s")

    @functools.partial(
        pl.kernel,
        mesh=mesh,
        out_type=(
            jax.ShapeDtypeStruct((B, _GW), jnp.float32),
            jax.ShapeDtypeStruct((B, _GW), jnp.float32),
        ),
        scratch_types=[
            pltpu.VMEM((6, b_per_w), jnp.int32),
            pltpu.VMEM((b_per_w,), jnp.int32),
            pltpu.VMEM((b_per_w,), jnp.int32),
            pltpu.VMEM((_HALF, _GW), jnp.float32),
            pltpu.SemaphoreType.DMA,
        ],
    )
    def gather_k(i0, i1, i2, i3, i4, i5, t1_hbm, t2_hbm, out1, out2,
                 idx_raw, idx1, idx2, rows, sem):
        wid = lax.axis_index("s") * _NC + lax.axis_index("c")
        base = wid * b_per_w
        cps = [pltpu.async_copy(i_hbm.at[wid], idx_raw.at[f], sem)
               for f, i_hbm in enumerate((i0, i1, i2, i3, i4, i5))]
        for cp in cps:
            cp.wait()

        @pl.loop(0, b_per_w // _LANE)
        def combine(g):
            sl = pl.ds(g * _LANE, _LANE)
            st = idx_raw[0, sl]
            yr = idx_raw[1, sl]
            mo = idx_raw[2, sl]
            dw = idx_raw[3, sl]
            hr = idx_raw[4, sl]
            se = idx_raw[5, sl]
            idx1[sl] = (st * 48 + yr * 12) + mo
            idx2[sl] = (dw * 96 + hr * 4) + se

        for h in range(b_per_w // _HALF):
            off = h * _HALF
            for tab, idxv, out in ((t1_hbm, idx1, out1), (t2_hbm, idx2, out2)):
                cps = [
                    pltpu.async_copy(
                        tab.at[idxv.at[pl.ds(off + c * _CHUNK, _CHUNK)]],
                        rows.at[pl.ds(c * _CHUNK, _CHUNK)], sem)
                    for c in range(_HALF // _CHUNK)
                ]
                for cp in cps:
                    cp.wait()
                pltpu.sync_copy(rows, out.at[pl.ds(base + off, _HALF)])

    return gather_k(*idxs, t1, t2)


def _mlp_body(e1, e2, w1, b1, w2, b2, w3, b3, w4, b4, w5, b5, out_ref):
    x = jnp.concatenate([e1[...], e2[...]], axis=1).astype(jnp.bfloat16)
    h = jnp.dot(x, w1[...], preferred_element_type=jnp.float32) + b1[...]
    h = jnp.maximum(h, 0.0).astype(jnp.bfloat16)
    h = jnp.dot(h, w2[...], preferred_element_type=jnp.float32) + b2[...]
    h = jnp.maximum(h, 0.0).astype(jnp.bfloat16)
    h = jnp.dot(h, w3[...], preferred_element_type=jnp.float32) + b3[...]
    h = jnp.maximum(h, 0.0).astype(jnp.bfloat16)
    h = jnp.dot(h, w4[...], preferred_element_type=jnp.float32) + b4[...]
    h = jnp.maximum(h, 0.0).astype(jnp.bfloat16)
    z = jnp.dot(h, w5[...], preferred_element_type=jnp.float32) + b5[...]
    out_ref[...] = 1.0 / (1.0 + jnp.exp(-z))


def _mlp(e1, e2, w1p, b1, w2, b2, w3, b3, w4, b4, w5, b5, tile=2048):
    B = e1.shape[0]
    full = lambda arr: pl.BlockSpec(arr.shape, lambda i: (0,) * arr.ndim)
    return pl.pallas_call(
        _mlp_body,
        grid=(B // tile,),
        in_specs=[
            pl.BlockSpec((tile, _GW), lambda i: (i, 0)),
            pl.BlockSpec((tile, _GW), lambda i: (i, 0)),
            full(w1p), full(b1), full(w2), full(b2),
            full(w3), full(b3), full(w4), full(b4),
            full(w5), full(b5),
        ],
        out_specs=pl.BlockSpec((tile, 1), lambda i: (i, 0)),
        out_shape=jax.ShapeDtypeStruct((B, 1), jnp.float32),
    )(e1, e2, w1p, b1, w2, b2, w3, b3, w4, b4, w5, b5)


def kernel(stations, year, month, day_of_week, hour, season,
           E_st, E_yr, E_mo, E_dw, E_hr, E_se,
           W1, b1, W2, b2, W3, b3, W4, b4, W5, b5):
    B = stations.shape[0]

    t1 = jnp.concatenate([
        jnp.broadcast_to(E_st[:, None, None, :], (33, 4, 12, 10)),
        jnp.broadcast_to(E_yr[None, :, None, :], (33, 4, 12, 2)),
        jnp.broadcast_to(E_mo[None, None, :, :], (33, 4, 12, 6)),
    ], axis=-1).reshape(1584, _D1)
    t1 = jnp.pad(t1, ((0, 0), (0, _GW - _D1)))
    t2 = jnp.concatenate([
        jnp.broadcast_to(E_dw[:, None, None, :], (7, 24, 4, 3)),
        jnp.broadcast_to(E_hr[None, :, None, :], (7, 24, 4, 10)),
        jnp.broadcast_to(E_se[None, None, :, :], (7, 24, 4, 2)),
    ], axis=-1).reshape(672, _D2)
    t2 = jnp.pad(t2, ((0, 0), (0, _GW - _D2)))

    bf = jnp.bfloat16
    w1p = jnp.zeros((2 * _GW, W1.shape[1]), bf)
    w1p = lax.dynamic_update_slice(w1p, W1[:_D1].astype(bf), (0, 0))
    w1p = lax.dynamic_update_slice(w1p, W1[_D1:].astype(bf), (_GW, 0))

    b_per_w = B // _NW
    idxs = [a.astype(jnp.int32).reshape(_NW, b_per_w) for a in
            (stations, year, month, day_of_week, hour, season)]
    e1, e2 = _sc_gather(idxs, t1, t2, B)

    return _mlp(e1, e2,
                w1p, b1.reshape(1, -1),
                W2.astype(bf), b2.reshape(1, -1),
                W3.astype(bf), b3.reshape(1, -1),
                W4.astype(bf), b4.reshape(1, -1),
                W5.astype(bf), b5.reshape(1, -1))
